# baseline (speedup 1.0000x reference)
import dataclasses
import functools

import jax
import jax.numpy as jnp
from jax import lax
from jax.experimental import pallas as pl
from jax.experimental.pallas import tpu as pltpu
from jax.experimental.pallas import tpu_sc as plsc

_V = 8192
_D = 1024
_L = 16
_NW = 32
_RPW = _V // _NW
_C = 32
_NCH = _RPW // _C
_NB = 3

_mesh = plsc.VectorSubcoreMesh(core_axis_name="c", subcore_axis_name="s")

_cp = pltpu.CompilerParams()
if "needs_layout_passes" in pltpu.CompilerParams.__dataclass_fields__:
    _cp = dataclasses.replace(_cp, needs_layout_passes=False)


def _copy_loop(src_slices, dst_slices, bufs, gsems, ssems):
    n = len(src_slices)
    nb = len(bufs)
    gather_cp = [None] * nb
    store_cp = [None] * nb
    for c in range(min(nb - 1, n)):
        gather_cp[c] = pltpu.async_copy(src_slices[c], bufs[c], gsems[c])
    for c in range(n):
        cur = c % nb
        if c + nb - 1 < n:
            pre = (c + nb - 1) % nb
            if store_cp[pre] is not None:
                store_cp[pre].wait()
                store_cp[pre] = None
            gather_cp[pre] = pltpu.async_copy(
                src_slices[c + nb - 1], bufs[pre], gsems[pre])
        gather_cp[cur].wait()
        store_cp[cur] = pltpu.async_copy(bufs[cur], dst_slices[c], ssems[cur])
    for b in range(nb):
        if store_cp[b] is not None:
            store_cp[b].wait()


@functools.partial(
    pl.kernel,
    out_type=jax.ShapeDtypeStruct((_V, _D), jnp.float32),
    mesh=_mesh,
    compiler_params=_cp,
)
def _sc_embed(table, limit_hbm, out):
    def body(lim_v, idx_v, b0, b1, b2, g0, g1, g2, s0, s1, s2):
        wid = lax.axis_index("s") * 2 + lax.axis_index("c")
        base = wid * _RPW

        pltpu.sync_copy(limit_hbm, lim_v)
        limit = lim_v[...]
        limit_s = jax.lax.reduce_max(limit, (0,))

        bufs = (b0, b1, b2)
        gsems = (g0, g1, g2)
        ssems = (s0, s1, s2)
        dsts = [out.at[pl.ds(base + c * _C, _C)] for c in range(_NCH)]

        @pl.when(limit_s >= _V - 1)
        def _fast():
            srcs = [table.at[pl.ds(base + c * _C, _C)] for c in range(_NCH)]
            _copy_loop(srcs, dsts, bufs, gsems, ssems)

        @pl.when(limit_s < _V - 1)
        def _general():
            ramp = lax.iota(jnp.int32, _L)
            for c in range(_NCH):
                for j in range(_C // _L):
                    idx_v[c, pl.ds(j * _L, _L)] = jnp.minimum(
                        ramp + (base + c * _C + j * _L), limit)
            srcs = [table.at[idx_v.at[c]] for c in range(_NCH)]
            _copy_loop(srcs, dsts, bufs, gsems, ssems)

    pl.run_scoped(
        body,
        pltpu.VMEM((_L,), jnp.int32),
        pltpu.VMEM((_NCH, _C), jnp.int32),
        pltpu.VMEM((_C, _D), jnp.float32),
        pltpu.VMEM((_C, _D), jnp.float32),
        pltpu.VMEM((_C, _D), jnp.float32),
        pltpu.SemaphoreType.DMA,
        pltpu.SemaphoreType.DMA,
        pltpu.SemaphoreType.DMA,
        pltpu.SemaphoreType.DMA,
        pltpu.SemaphoreType.DMA,
        pltpu.SemaphoreType.DMA,
    )


def kernel(posit_embedding, seq_length):
    s = jnp.asarray(seq_length, jnp.int32)
    limit = jnp.clip(s - 1, 0, _V - 1)
    limit_vec = jnp.broadcast_to(limit, (_L,)).astype(jnp.int32)
    out = _sc_embed(posit_embedding, limit_vec)
    return out[None, :, :]
